# baseline (speedup 1.0000x reference)
import functools

import jax
import jax.numpy as jnp
from jax import lax
from jax.experimental import pallas as pl
from jax.experimental.pallas import tpu as pltpu
from jax.experimental.pallas import tpu_sc as plsc

N_NODES = 10000
N_EDGES = 320000
DIM = 128

_NC = 2
_NS = 16
_NW = _NC * _NS
_EPW = N_EDGES // _NW
_C = 80
_K = _EPW // _C
_BROWS = 80
_NB = N_NODES // _BROWS


def _agg_body(row_hbm, col_hbm, w_hbm, emb_hbm, out_hbm,
              rowv0, rowv1, rowv2, rowv3, colv0, colv1, colv2, colv3,
              wv0, wv1, wv2, wv3, gbuf0, gbuf1, sbuf0, sbuf1, acc,
              gsem0, gsem1, ssem0, ssem1, isem0, isem1):
    c = lax.axis_index("c")
    s = lax.axis_index("s")
    wid = c * _NS + s
    base = wid * _EPW

    rowvs = (rowv0, rowv1, rowv2, rowv3)
    colvs = (colv0, colv1, colv2, colv3)
    wvs = (wv0, wv1, wv2, wv3)
    gbufs = (gbuf0, gbuf1)
    sbufs = (sbuf0, sbuf1)
    gsems = (gsem0, gsem1)
    ssems = (ssem0, ssem1)
    isems = (isem0, isem1)

    def zfill(i, carry):
        for j in range(DIM // 16):
            sbuf0[i, pl.ds(16 * j, 16)] = jnp.zeros((16,), jnp.float32)
        return carry

    lax.fori_loop(0, _BROWS, zfill, 0)

    def zblock(i, carry):
        b = s + _NS * i

        @pl.when(b < _NB)
        def _():
            off = pl.multiple_of(b * _BROWS, _BROWS)
            pltpu.sync_copy(sbuf0, acc.at[pl.ds(off, _BROWS)])

        return carry

    lax.fori_loop(0, (_NB + _NS - 1) // _NS, zblock, 0)

    plsc.subcore_barrier()

    def idx_issue(k, q, p):
        off = pl.multiple_of(base + k * _C, _C)
        pltpu.async_copy(row_hbm.at[pl.ds(off, _C)], rowvs[q], isems[p])
        pltpu.async_copy(col_hbm.at[pl.ds(off, _C)], colvs[q], isems[p])
        pltpu.async_copy(w_hbm.at[pl.ds(off, _C)], wvs[q], isems[p])

    def idx_drain(k, q, p):
        off = pl.multiple_of(base + k * _C, _C)
        pltpu.make_async_copy(row_hbm.at[pl.ds(off, _C)], rowvs[q],
                              isems[p]).wait()
        pltpu.make_async_copy(col_hbm.at[pl.ds(off, _C)], colvs[q],
                              isems[p]).wait()
        pltpu.make_async_copy(w_hbm.at[pl.ds(off, _C)], wvs[q],
                              isems[p]).wait()

    def half(k, q):
        p = q % 2

        @pl.when(k >= 2)
        def _():
            pltpu.make_async_copy(sbufs[p], acc.at[rowvs[(q + 2) % 4]],
                                  ssems[p]).wait()

        @pl.when(k + 2 < _K)
        def _():
            idx_issue(k + 2, (q + 2) % 4, p)

        pltpu.make_async_copy(emb_hbm.at[colvs[q]], gbufs[p],
                              gsems[p]).wait()

        def scale_group(g, carry2):
            w16 = wvs[q][pl.ds(g * 16, 16)]
            for l in range(16):
                bw = jnp.broadcast_to(w16[l], (16,))
                e = g * 16 + l
                hi_mask = jnp.full((16,), -65536, jnp.int32)
                for j in range(DIM // 32):
                    xi = gbufs[p][e, pl.ds(16 * j, 16)]
                    ev = lax.bitcast_convert_type(xi << 16, jnp.float32)
                    od = lax.bitcast_convert_type(xi & hi_mask, jnp.float32)
                    sbufs[p][e, pl.ds(32 * j, 16)] = ev * bw
                    sbufs[p][e, pl.ds(32 * j + 16, 16)] = od * bw
            return carry2

        lax.fori_loop(0, _C // 16, scale_group, 0)

        pltpu.async_copy(sbufs[p], acc.at[rowvs[q]], ssems[p], add=True)

        @pl.when(k + 2 < _K)
        def _():
            idx_drain(k + 2, (q + 2) % 4, p)
            pltpu.async_copy(emb_hbm.at[colvs[(q + 2) % 4]], gbufs[p],
                             gsems[p])

    idx_issue(0, 0, 0)
    idx_issue(1, 1, 1)
    idx_drain(0, 0, 0)
    idx_drain(1, 1, 1)
    pltpu.async_copy(emb_hbm.at[colv0], gbuf0, gsem0)
    pltpu.async_copy(emb_hbm.at[colv1], gbuf1, gsem1)

    def quad(t, carry):
        k0 = 4 * t
        half(k0, 0)
        half(k0 + 1, 1)
        half(k0 + 2, 2)
        half(k0 + 3, 3)
        return carry

    lax.fori_loop(0, _K // 4, quad, 0)
    half(_K - 1, 0)

    pltpu.make_async_copy(sbufs[1], acc.at[rowvs[3]], ssems[1]).wait()
    pltpu.make_async_copy(sbufs[0], acc.at[rowvs[0]], ssems[0]).wait()

    plsc.subcore_barrier()

    def wblock(i, carry):
        b = s + _NS * i

        @pl.when(b < _NB)
        def _():
            off = pl.multiple_of(b * _BROWS, _BROWS)
            pltpu.sync_copy(acc.at[pl.ds(off, _BROWS)],
                            out_hbm.at[c, pl.ds(off, _BROWS)])

        return carry

    lax.fori_loop(0, (_NB + _NS - 1) // _NS, wblock, 0)


_aggregate = functools.partial(
    pl.kernel,
    out_type=jax.ShapeDtypeStruct((_NC, N_NODES, DIM), jnp.float32),
    mesh=plsc.VectorSubcoreMesh(core_axis_name="c", subcore_axis_name="s"),
    compiler_params=pltpu.CompilerParams(use_tc_tiling_on_sc=False),
    scratch_types=(
        [pltpu.VMEM((_C,), jnp.int32) for _ in range(8)]
        + [pltpu.VMEM((_C,), jnp.float32) for _ in range(4)]
        + [pltpu.VMEM((_C, DIM // 2), jnp.int32) for _ in range(2)]
        + [pltpu.VMEM((_C, DIM), jnp.float32) for _ in range(2)]
        + [pltpu.VMEM_SHARED((N_NODES, DIM), jnp.float32)]
        + [pltpu.SemaphoreType.DMA for _ in range(6)]
    ),
)(_agg_body)


_BM = 1000


def _mlp_body(p_ref, w1_ref, w2_ref, o_ref):
    x = p_ref[0] + p_ref[1]
    h = jnp.maximum(
        lax.dot_general(x, w1_ref[...], (((1,), (1,)), ((), ())),
                        preferred_element_type=jnp.float32), 0.0)
    o_ref[...] = lax.dot_general(h, w2_ref[...], (((1,), (1,)), ((), ())),
                                 preferred_element_type=jnp.float32)


def _mlp(partials, W1, W2):
    return pl.pallas_call(
        _mlp_body,
        grid=(N_NODES // _BM,),
        in_specs=[
            pl.BlockSpec((_NC, _BM, DIM), lambda i: (0, i, 0)),
            pl.BlockSpec((DIM, DIM), lambda i: (0, 0)),
            pl.BlockSpec((DIM, DIM), lambda i: (0, 0)),
        ],
        out_specs=pl.BlockSpec((_BM, DIM), lambda i: (i, 0)),
        out_shape=jax.ShapeDtypeStruct((N_NODES, DIM), jnp.float32),
    )(partials, W1, W2)


_PERM = []
for _j in range(DIM // 32):
    _PERM.extend(32 * _j + 2 * _i for _i in range(16))
    _PERM.extend(32 * _j + 2 * _i + 1 for _i in range(16))


@jax.jit
def kernel(edge_index, edge_weight, emb_weight, W1, W2):
    emb_bf = emb_weight.astype(jnp.bfloat16)
    emb_i = lax.bitcast_convert_type(
        emb_bf.reshape(N_NODES, DIM // 2, 2), jnp.int32)
    w1p = W1[:, jnp.array(_PERM, dtype=jnp.int32)]
    partials = _aggregate(edge_index[0], edge_index[1], edge_weight, emb_i)
    return _mlp(partials, w1p, W2)


# speedup vs baseline: 1.6856x; 1.6856x over previous
import functools

import jax
import jax.numpy as jnp
from jax import lax
from jax.experimental import pallas as pl
from jax.experimental.pallas import tpu as pltpu
from jax.experimental.pallas import tpu_sc as plsc

N_NODES = 10000
N_EDGES = 320000
DIM = 128

_NC = 2
_NS = 16
_NW = _NC * _NS
_EPW = N_EDGES // _NW
_C = 80
_K = _EPW // _C
_BROWS = 80
_NB = N_NODES // _BROWS


def _agg_body(row_hbm, col_hbm, w_hbm, emb_hbm, out_hbm,
              rowv0, rowv1, rowv2, rowv3, colv0, colv1, colv2, colv3,
              wv0, wv1, wv2, wv3, gbuf0, gbuf1, sbuf0, sbuf1, acc,
              gsem0, gsem1, ssem0, ssem1, isem0, isem1):
    c = lax.axis_index("c")
    s = lax.axis_index("s")
    wid = c * _NS + s
    base = wid * _EPW

    rowvs = (rowv0, rowv1, rowv2, rowv3)
    colvs = (colv0, colv1, colv2, colv3)
    wvs = (wv0, wv1, wv2, wv3)
    gbufs = (gbuf0, gbuf1)
    sbufs = (sbuf0, sbuf1)
    gsems = (gsem0, gsem1)
    ssems = (ssem0, ssem1)
    isems = (isem0, isem1)

    def zfill(i, carry):
        for j in range(DIM // 16):
            sbuf0[i, pl.ds(16 * j, 16)] = jnp.zeros((16,), jnp.float32)
        return carry

    lax.fori_loop(0, _BROWS, zfill, 0)

    def zblock(i, carry):
        b = s + _NS * i

        @pl.when(b < _NB)
        def _():
            off = pl.multiple_of(b * _BROWS, _BROWS)
            pltpu.sync_copy(sbuf0, acc.at[pl.ds(off, _BROWS)])

        return carry

    lax.fori_loop(0, (_NB + _NS - 1) // _NS, zblock, 0)

    plsc.subcore_barrier()

    def idx_issue(k, q, p):
        off = pl.multiple_of(base + k * _C, _C)
        pltpu.async_copy(row_hbm.at[pl.ds(off, _C)], rowvs[q], isems[p])
        pltpu.async_copy(col_hbm.at[pl.ds(off, _C)], colvs[q], isems[p])
        pltpu.async_copy(w_hbm.at[pl.ds(off, _C)], wvs[q], isems[p])

    def idx_drain(k, q, p):
        off = pl.multiple_of(base + k * _C, _C)
        pltpu.make_async_copy(row_hbm.at[pl.ds(off, _C)], rowvs[q],
                              isems[p]).wait()
        pltpu.make_async_copy(col_hbm.at[pl.ds(off, _C)], colvs[q],
                              isems[p]).wait()
        pltpu.make_async_copy(w_hbm.at[pl.ds(off, _C)], wvs[q],
                              isems[p]).wait()

    def half(k, q):
        p = q % 2

        @pl.when(k >= 2)
        def _():
            pltpu.make_async_copy(sbufs[p], acc.at[rowvs[(q + 2) % 4]],
                                  ssems[p]).wait()

        @pl.when(k + 2 < _K)
        def _():
            idx_issue(k + 2, (q + 2) % 4, p)

        pltpu.make_async_copy(emb_hbm.at[colvs[q]], gbufs[p],
                              gsems[p]).wait()

        @plsc.parallel_loop(0, _C // 16)
        def scale_group(g):
            w16 = wvs[q][pl.ds(g * 16, 16)]
            for l in range(16):
                bw = jnp.broadcast_to(w16[l], (16,))
                e = g * 16 + l
                for j in range(DIM // 16):
                    sbufs[p][e, pl.ds(16 * j, 16)] = (
                        gbufs[p][e, pl.ds(16 * j, 16)] * bw)

        pltpu.async_copy(sbufs[p], acc.at[rowvs[q]], ssems[p], add=True)

        @pl.when(k + 2 < _K)
        def _():
            idx_drain(k + 2, (q + 2) % 4, p)
            pltpu.async_copy(emb_hbm.at[colvs[(q + 2) % 4]], gbufs[p],
                             gsems[p])

    idx_issue(0, 0, 0)
    idx_issue(1, 1, 1)
    idx_drain(0, 0, 0)
    idx_drain(1, 1, 1)
    pltpu.async_copy(emb_hbm.at[colv0], gbuf0, gsem0)
    pltpu.async_copy(emb_hbm.at[colv1], gbuf1, gsem1)

    def quad(t, carry):
        k0 = 4 * t
        half(k0, 0)
        half(k0 + 1, 1)
        half(k0 + 2, 2)
        half(k0 + 3, 3)
        return carry

    lax.fori_loop(0, _K // 4, quad, 0)
    half(_K - 1, 0)

    pltpu.make_async_copy(sbufs[1], acc.at[rowvs[3]], ssems[1]).wait()
    pltpu.make_async_copy(sbufs[0], acc.at[rowvs[0]], ssems[0]).wait()

    plsc.subcore_barrier()

    def wblock(i, carry):
        b = s + _NS * i

        @pl.when(b < _NB)
        def _():
            off = pl.multiple_of(b * _BROWS, _BROWS)
            pltpu.sync_copy(acc.at[pl.ds(off, _BROWS)],
                            out_hbm.at[c, pl.ds(off, _BROWS)])

        return carry

    lax.fori_loop(0, (_NB + _NS - 1) // _NS, wblock, 0)


_aggregate = functools.partial(
    pl.kernel,
    out_type=jax.ShapeDtypeStruct((_NC, N_NODES, DIM), jnp.float32),
    mesh=plsc.VectorSubcoreMesh(core_axis_name="c", subcore_axis_name="s"),
    scratch_types=(
        [pltpu.VMEM((_C,), jnp.int32) for _ in range(8)]
        + [pltpu.VMEM((_C,), jnp.float32) for _ in range(4)]
        + [pltpu.VMEM((_C, DIM), jnp.float32) for _ in range(4)]
        + [pltpu.VMEM_SHARED((N_NODES, DIM), jnp.float32)]
        + [pltpu.SemaphoreType.DMA for _ in range(6)]
    ),
)(_agg_body)


_BM = 1000


def _mlp_body(p_ref, w1_ref, w2_ref, o_ref):
    x = p_ref[0] + p_ref[1]
    h = jnp.maximum(
        lax.dot_general(x, w1_ref[...], (((1,), (1,)), ((), ())),
                        preferred_element_type=jnp.float32), 0.0)
    o_ref[...] = lax.dot_general(h, w2_ref[...], (((1,), (1,)), ((), ())),
                                 preferred_element_type=jnp.float32)


def _mlp(partials, W1, W2):
    return pl.pallas_call(
        _mlp_body,
        grid=(N_NODES // _BM,),
        in_specs=[
            pl.BlockSpec((_NC, _BM, DIM), lambda i: (0, i, 0)),
            pl.BlockSpec((DIM, DIM), lambda i: (0, 0)),
            pl.BlockSpec((DIM, DIM), lambda i: (0, 0)),
        ],
        out_specs=pl.BlockSpec((_BM, DIM), lambda i: (i, 0)),
        out_shape=jax.ShapeDtypeStruct((N_NODES, DIM), jnp.float32),
    )(partials, W1, W2)


@jax.jit
def kernel(edge_index, edge_weight, emb_weight, W1, W2):
    partials = _aggregate(edge_index[0], edge_index[1], edge_weight,
                          emb_weight)
    return _mlp(partials, W1, W2)
